# baseline (speedup 1.0000x reference)
import jax
import jax.numpy as jnp
from jax.experimental import pallas as pl
from jax.experimental.pallas import tpu as pltpu

NCB = 16
CB_DIM = 4
CB_SIZE = 1024
_INV_LN2 = 1.4426950408889634


def _vq_kernel(lam_ref, xs_ref, cbTn_ref, aug_ref, logits_ref,
               xhat_ref, rate_ref):
    c = pl.program_id(0)
    nb = pl.program_id(1)

    @pl.when((c == 0) & (nb == 0))
    def _init():
        rate_ref[0, 0] = 0.0

    xs = xs_ref[0]
    cbTn = cbTn_ref[0]
    lg = logits_ref[0]

    m = jnp.max(lg, axis=-1, keepdims=True)
    lse = jnp.log(jnp.sum(jnp.exp(lg - m), axis=-1, keepdims=True)) + m
    log2p = (lse - lg) * _INV_LN2
    rate_bias = log2p / lam_ref[0, 0]

    xn = jnp.sum(xs * xs, axis=0)[:, None]
    cbn = 0.25 * jnp.sum(cbTn * cbTn, axis=0)[None, :]

    prodn = jax.lax.dot_general(xs, cbTn, (((0,), (0,)), ((), ())),
                                preferred_element_type=jnp.float32)
    dist = rate_bias + ((xn + cbn) + prodn)

    idx = jnp.argmin(dist, axis=1)
    iota = jax.lax.broadcasted_iota(jnp.int32, dist.shape, 1)
    oh = (iota == idx[:, None]).astype(jnp.float32)

    g = jax.lax.dot_general(oh, aug_ref[0], (((1,), (0,)), ((), ())),
                            preferred_element_type=jnp.float32)
    xhat_ref[0] = g[:, :CB_DIM]
    rate_ref[0, 0] += jnp.sum(g[:, CB_DIM])


def kernel(x, codebook, logits, lmbda):
    shape = x.shape
    xf = x.reshape(-1, NCB, CB_DIM)
    n = xf.shape[0]
    nb = min(n, 512)
    nblk = n // nb

    xs = xf.transpose(1, 2, 0)
    cbTn = codebook.transpose(0, 2, 1) * (-2.0)
    lg3 = logits.reshape(NCB, 1, CB_SIZE)
    lam = jnp.asarray(lmbda, jnp.float32).reshape(1, 1)
    log2p_col = jax.nn.log_softmax(logits, axis=-1) * (-_INV_LN2)
    aug = jnp.concatenate(
        [codebook, log2p_col[..., None],
         jnp.zeros((NCB, CB_SIZE, 3), jnp.float32)], axis=-1)

    xhat_t, rate = pl.pallas_call(
        _vq_kernel,
        grid=(NCB, nblk),
        in_specs=[
            pl.BlockSpec(memory_space=pltpu.SMEM),
            pl.BlockSpec((1, CB_DIM, nb), lambda c, b: (c, 0, b)),
            pl.BlockSpec((1, CB_DIM, CB_SIZE), lambda c, b: (c, 0, 0)),
            pl.BlockSpec((1, CB_SIZE, 8), lambda c, b: (c, 0, 0)),
            pl.BlockSpec((1, 1, CB_SIZE), lambda c, b: (c, 0, 0)),
        ],
        out_specs=[
            pl.BlockSpec((1, nb, CB_DIM), lambda c, b: (c, b, 0)),
            pl.BlockSpec(memory_space=pltpu.SMEM),
        ],
        out_shape=[
            jax.ShapeDtypeStruct((NCB, n, CB_DIM), jnp.float32),
            jax.ShapeDtypeStruct((1, 1), jnp.float32),
        ],
    )(lam, xs, cbTn, aug, lg3)

    x_hat = xhat_t.transpose(1, 0, 2).reshape(shape)
    rate_uem = rate[0, 0]
    zero = jnp.zeros((1,), dtype=jnp.float32)
    return (x_hat, rate_uem, jnp.zeros_like(rate_uem), zero, zero)


# speedup vs baseline: 1.1177x; 1.1177x over previous
import jax
import jax.numpy as jnp
from jax.experimental import pallas as pl
from jax.experimental.pallas import tpu as pltpu

NCB = 16
CB_DIM = 4
CB_SIZE = 1024
_INV_LN2 = 1.4426950408889634


def _vq_kernel(lam_ref, xs_ref, cbTn_ref, aug_ref, logits_ref,
               xhat_ref, rate_ref):
    c = pl.program_id(0)
    nb = pl.program_id(1)

    @pl.when((c == 0) & (nb == 0))
    def _init():
        rate_ref[0, 0] = 0.0

    xs = xs_ref[0]
    cbTn = cbTn_ref[0]
    lg = logits_ref[0]

    m = jnp.max(lg, axis=-1, keepdims=True)
    lse = jnp.log(jnp.sum(jnp.exp(lg - m), axis=-1, keepdims=True)) + m
    log2p = (lse - lg) * _INV_LN2
    rate_bias = log2p / lam_ref[0, 0]

    xn = jnp.sum(xs * xs, axis=0)[:, None]
    cbn = 0.25 * jnp.sum(cbTn * cbTn, axis=0)[None, :]

    prodn = jax.lax.dot_general(xs, cbTn, (((0,), (0,)), ((), ())),
                                preferred_element_type=jnp.float32)
    dist = rate_bias + ((xn + cbn) + prodn)

    idx = jnp.argmin(dist, axis=1)
    iota = jax.lax.broadcasted_iota(jnp.int32, dist.shape, 1)
    oh = (iota == idx[:, None]).astype(jnp.float32)

    g = jax.lax.dot_general(oh, aug_ref[0], (((1,), (0,)), ((), ())),
                            preferred_element_type=jnp.float32)
    xhat_ref[0] = g[:, :CB_DIM]
    rate_ref[0, 0] += jnp.sum(g[:, CB_DIM])


def kernel(x, codebook, logits, lmbda):
    shape = x.shape
    xf = x.reshape(-1, NCB, CB_DIM)
    n = xf.shape[0]
    nb = min(n, 1024)
    nblk = n // nb

    xs = xf.transpose(1, 2, 0)
    cbTn = codebook.transpose(0, 2, 1) * (-2.0)
    lg3 = logits.reshape(NCB, 1, CB_SIZE)
    lam = jnp.asarray(lmbda, jnp.float32).reshape(1, 1)
    log2p_col = jax.nn.log_softmax(logits, axis=-1) * (-_INV_LN2)
    aug = jnp.concatenate(
        [codebook, log2p_col[..., None],
         jnp.zeros((NCB, CB_SIZE, 3), jnp.float32)], axis=-1)

    xhat_t, rate = pl.pallas_call(
        _vq_kernel,
        grid=(NCB, nblk),
        in_specs=[
            pl.BlockSpec(memory_space=pltpu.SMEM),
            pl.BlockSpec((1, CB_DIM, nb), lambda c, b: (c, 0, b)),
            pl.BlockSpec((1, CB_DIM, CB_SIZE), lambda c, b: (c, 0, 0)),
            pl.BlockSpec((1, CB_SIZE, 8), lambda c, b: (c, 0, 0)),
            pl.BlockSpec((1, 1, CB_SIZE), lambda c, b: (c, 0, 0)),
        ],
        out_specs=[
            pl.BlockSpec((1, nb, CB_DIM), lambda c, b: (c, b, 0)),
            pl.BlockSpec(memory_space=pltpu.SMEM),
        ],
        out_shape=[
            jax.ShapeDtypeStruct((NCB, n, CB_DIM), jnp.float32),
            jax.ShapeDtypeStruct((1, 1), jnp.float32),
        ],
    )(lam, xs, cbTn, aug, lg3)

    x_hat = xhat_t.transpose(1, 0, 2).reshape(shape)
    rate_uem = rate[0, 0]
    zero = jnp.zeros((1,), dtype=jnp.float32)
    return (x_hat, rate_uem, jnp.zeros_like(rate_uem), zero, zero)


# speedup vs baseline: 1.2900x; 1.1541x over previous
import jax
import jax.numpy as jnp
from jax.experimental import pallas as pl
from jax.experimental.pallas import tpu as pltpu

NCB = 16
CB_DIM = 4
CB_SIZE = 1024
_INV_LN2 = 1.4426950408889634


def _vq_kernel(lam_ref, xs_ref, cbTn_ref, aug_ref, logits_ref,
               xhat_ref, rate_ref):
    c = pl.program_id(0)
    nb = pl.program_id(1)

    @pl.when((c == 0) & (nb == 0))
    def _init():
        rate_ref[0, 0] = 0.0

    xs = xs_ref[0]
    cbTn = cbTn_ref[0]
    lg = logits_ref[0]

    m = jnp.max(lg, axis=-1, keepdims=True)
    lse = jnp.log(jnp.sum(jnp.exp(lg - m), axis=-1, keepdims=True)) + m
    log2p = (lse - lg) * _INV_LN2
    rate_bias = log2p / lam_ref[0, 0]

    xn = jnp.sum(xs * xs, axis=0)[:, None]
    cbn = 0.25 * jnp.sum(cbTn * cbTn, axis=0)[None, :]

    prodn = jax.lax.dot_general(xs, cbTn, (((0,), (0,)), ((), ())),
                                preferred_element_type=jnp.float32)
    dist = rate_bias + ((xn + cbn) + prodn)

    idx = jnp.argmin(dist, axis=1)
    iota = jax.lax.broadcasted_iota(jnp.int32, dist.shape, 1)
    oh = (iota == idx[:, None]).astype(jnp.float32)

    g = jax.lax.dot_general(oh, aug_ref[0], (((1,), (0,)), ((), ())),
                            preferred_element_type=jnp.float32)
    xhat_ref[0] = g[:, :CB_DIM]
    rate_ref[0, 0] += jnp.sum(g[:, CB_DIM])


def kernel(x, codebook, logits, lmbda):
    shape = x.shape
    xf = x.reshape(-1, NCB, CB_DIM)
    n = xf.shape[0]
    nb = min(n, 2048)
    nblk = n // nb

    xs = xf.transpose(1, 2, 0)
    cbTn = codebook.transpose(0, 2, 1) * (-2.0)
    lg3 = logits.reshape(NCB, 1, CB_SIZE)
    lam = jnp.asarray(lmbda, jnp.float32).reshape(1, 1)
    log2p_col = jax.nn.log_softmax(logits, axis=-1) * (-_INV_LN2)
    aug = jnp.concatenate(
        [codebook, log2p_col[..., None],
         jnp.zeros((NCB, CB_SIZE, 3), jnp.float32)], axis=-1)

    xhat_t, rate = pl.pallas_call(
        _vq_kernel,
        grid=(NCB, nblk),
        in_specs=[
            pl.BlockSpec(memory_space=pltpu.SMEM),
            pl.BlockSpec((1, CB_DIM, nb), lambda c, b: (c, 0, b)),
            pl.BlockSpec((1, CB_DIM, CB_SIZE), lambda c, b: (c, 0, 0)),
            pl.BlockSpec((1, CB_SIZE, 8), lambda c, b: (c, 0, 0)),
            pl.BlockSpec((1, 1, CB_SIZE), lambda c, b: (c, 0, 0)),
        ],
        out_specs=[
            pl.BlockSpec((1, nb, CB_DIM), lambda c, b: (c, b, 0)),
            pl.BlockSpec(memory_space=pltpu.SMEM),
        ],
        out_shape=[
            jax.ShapeDtypeStruct((NCB, n, CB_DIM), jnp.float32),
            jax.ShapeDtypeStruct((1, 1), jnp.float32),
        ],
    )(lam, xs, cbTn, aug, lg3)

    x_hat = xhat_t.transpose(1, 0, 2).reshape(shape)
    rate_uem = rate[0, 0]
    zero = jnp.zeros((1,), dtype=jnp.float32)
    return (x_hat, rate_uem, jnp.zeros_like(rate_uem), zero, zero)


# speedup vs baseline: 1.4581x; 1.1303x over previous
import jax
import jax.numpy as jnp
from jax.experimental import pallas as pl
from jax.experimental.pallas import tpu as pltpu

NCB = 16
CB_DIM = 4
CB_SIZE = 1024
_INV_LN2 = 1.4426950408889634


def _vq_kernel(lam_ref, xs_ref, cbTn_ref, aug_ref, logits_ref,
               xhat_ref, rate_ref):
    c = pl.program_id(0)
    nb = pl.program_id(1)

    @pl.when((c == 0) & (nb == 0))
    def _init():
        rate_ref[0, 0] = 0.0

    xs = xs_ref[0]
    cbTn = cbTn_ref[0]
    lg = logits_ref[0]

    m = jnp.max(lg, axis=-1, keepdims=True)
    lse = jnp.log(jnp.sum(jnp.exp(lg - m), axis=-1, keepdims=True)) + m
    log2p = (lse - lg) * _INV_LN2
    rate_bias = log2p / lam_ref[0, 0]

    xn = jnp.sum(xs * xs, axis=0)[:, None]
    cbn = 0.25 * jnp.sum(cbTn * cbTn, axis=0)[None, :]

    prodn = jax.lax.dot_general(xs, cbTn, (((0,), (0,)), ((), ())),
                                preferred_element_type=jnp.float32)
    dist = rate_bias + ((xn + cbn) + prodn)

    idx = jnp.argmin(dist, axis=1)
    iota = jax.lax.broadcasted_iota(jnp.int32, dist.shape, 1)
    oh = (iota == idx[:, None]).astype(jnp.float32)

    g = jax.lax.dot_general(oh, aug_ref[0], (((1,), (0,)), ((), ())),
                            preferred_element_type=jnp.float32)
    xhat_ref[0] = g[:, :CB_DIM]
    rate_ref[0, 0] += jnp.sum(g[:, CB_DIM])


def kernel(x, codebook, logits, lmbda):
    shape = x.shape
    xf = x.reshape(-1, NCB, CB_DIM)
    n = xf.shape[0]
    nb = min(n, 4096)
    nblk = n // nb

    xs = xf.transpose(1, 2, 0)
    cbTn = codebook.transpose(0, 2, 1) * (-2.0)
    lg3 = logits.reshape(NCB, 1, CB_SIZE)
    lam = jnp.asarray(lmbda, jnp.float32).reshape(1, 1)
    log2p_col = jax.nn.log_softmax(logits, axis=-1) * (-_INV_LN2)
    aug = jnp.concatenate(
        [codebook, log2p_col[..., None],
         jnp.zeros((NCB, CB_SIZE, 3), jnp.float32)], axis=-1)

    xhat_t, rate = pl.pallas_call(
        _vq_kernel,
        grid=(NCB, nblk),
        in_specs=[
            pl.BlockSpec(memory_space=pltpu.SMEM),
            pl.BlockSpec((1, CB_DIM, nb), lambda c, b: (c, 0, b)),
            pl.BlockSpec((1, CB_DIM, CB_SIZE), lambda c, b: (c, 0, 0)),
            pl.BlockSpec((1, CB_SIZE, 8), lambda c, b: (c, 0, 0)),
            pl.BlockSpec((1, 1, CB_SIZE), lambda c, b: (c, 0, 0)),
        ],
        out_specs=[
            pl.BlockSpec((1, nb, CB_DIM), lambda c, b: (c, b, 0)),
            pl.BlockSpec(memory_space=pltpu.SMEM),
        ],
        out_shape=[
            jax.ShapeDtypeStruct((NCB, n, CB_DIM), jnp.float32),
            jax.ShapeDtypeStruct((1, 1), jnp.float32),
        ],
    )(lam, xs, cbTn, aug, lg3)

    x_hat = xhat_t.transpose(1, 0, 2).reshape(shape)
    rate_uem = rate[0, 0]
    zero = jnp.zeros((1,), dtype=jnp.float32)
    return (x_hat, rate_uem, jnp.zeros_like(rate_uem), zero, zero)
